# baseline (speedup 1.0000x reference)
import jax
import jax.numpy as jnp
from jax import lax
from jax.experimental import pallas as pl
from jax.experimental.pallas import tpu as pltpu
from jax.experimental.pallas import tpu_sc as plsc

_PCODE = 2380
_NROW = 2 * (_PCODE + 1)
_D = 64
_B, _H, _W = 1024, 19, 19
_HW = _H * _W
_HWP = 368
_NV = _HWP // 16
_NC, _NS = 2, 16
_NW = _NC * _NS
_NG = 4
_DG = _D // _NG
_NSG = _NW // _NG
_SPT = _B // _NSG
_TILE = _DG * _HW


def _body(prep, table_r, out,
          table_v, in0_v, in1_v, outt0_v, outt1_v,
          sem_t, sem_in0, sem_in1, sem_out):
    wid = lax.axis_index("s") * _NC + lax.axis_index("c")
    g = wid % _NG
    sg = wid // _NG
    base = sg * _SPT
    lanes = lax.iota(jnp.int32, 16)
    tail_mask = lanes < (_HW - 16 * (_NV - 1))
    ins = (in0_v, in1_v)
    outs = (outt0_v, outt1_v)
    sems = (sem_in0, sem_in1)

    pltpu.async_copy(table_r.at[g], table_v, sem_t)
    pltpu.async_copy(prep.at[base], ins[0], sem_in0)
    pltpu.async_copy(prep.at[base + 1], ins[1], sem_in1)
    pltpu.make_async_copy(table_r.at[g], table_v, sem_t).wait()

    def compute(in_ref, out_ref):
        for i in range(_NV):
            sl = pl.ds(16 * i, 16)
            s0 = in_ref[0, sl]
            s1 = in_ref[1, sl]
            b0 = plsc.bitcast(in_ref[2, sl], jnp.float32)
            b1 = plsc.bitcast(in_ref[3, sl], jnp.float32)
            i0 = jnp.where(b0 > 0.0, _PCODE, s0)
            i1 = jnp.where(b1 > 0.0, _PCODE + _PCODE + 1, s1 + (_PCODE + 1))
            a0 = i0 * _DG
            a1 = i1 * _DG
            mask = tail_mask if i == _NV - 1 else None
            for d in range(_DG):
                v0 = plsc.load_gather(table_v, [a0 + d])
                v1 = plsc.load_gather(table_v, [a1 + d])
                plsc.store_scatter(out_ref, [lanes + (d * _HW + 16 * i)],
                                   v0 + v1, mask=mask)

    @pl.loop(0, _SPT, step=2)
    def _pair(s0):
        for par in range(2):
            s = s0 + par
            b = base + s
            pltpu.make_async_copy(prep.at[b], ins[par], sems[par]).wait()
            compute(ins[par], outs[par])
            @pl.when(s > 0)
            def _():
                pltpu.make_async_copy(outs[par], out.at[b, g], sem_out).wait()
            pltpu.async_copy(outs[par], out.at[b, g], sem_out)
            pltpu.async_copy(prep.at[jnp.minimum(b + 2, _B - 1)],
                             ins[par], sems[par])

    pltpu.make_async_copy(outt0_v, out.at[base, g], sem_out).wait()
    pltpu.make_async_copy(prep.at[base], in0_v, sem_in0).wait()
    pltpu.make_async_copy(prep.at[base], in1_v, sem_in1).wait()


@jax.jit
def _pcode_embed(prep, table_r):
    mesh = plsc.VectorSubcoreMesh(core_axis_name="c", subcore_axis_name="s",
                                  num_cores=_NC, num_subcores=_NS)
    f = pl.kernel(
        _body,
        out_type=jax.ShapeDtypeStruct((_B, _NG, _TILE), jnp.float32),
        mesh=mesh,
        compiler_params=pltpu.CompilerParams(needs_layout_passes=False,
                                             use_tc_tiling_on_sc=False),
        scratch_types=[
            pltpu.VMEM((_NROW * _DG,), jnp.float32),
            pltpu.VMEM((4, _HWP), jnp.int32),
            pltpu.VMEM((4, _HWP), jnp.int32),
            pltpu.VMEM((_TILE,), jnp.float32),
            pltpu.VMEM((_TILE,), jnp.float32),
            pltpu.SemaphoreType.DMA,
            pltpu.SemaphoreType.DMA,
            pltpu.SemaphoreType.DMA,
            pltpu.SemaphoreType.DMA,
        ],
    )
    return f(prep, table_r)


def kernel(sparse_feature_input, board_input, sparse_feature_dim, pcode_table):
    del sparse_feature_dim
    pad = ((0, 0), (0, 0), (0, _HWP - _HW))
    sf = sparse_feature_input.reshape(_B, 12, _HW)[:, 10:12]
    bd = board_input.reshape(_B, 2, _HW).view(jnp.int32)
    prep = jnp.pad(jnp.concatenate([sf, bd], axis=1), pad)
    table_r = pcode_table.reshape(_NROW, _NG, _DG).transpose(1, 0, 2)
    table_r = table_r.reshape(_NG, _NROW * _DG)
    out = _pcode_embed(prep, table_r)
    return out.reshape(_B, _D, _H, _W)


# speedup vs baseline: 1.4074x; 1.4074x over previous
import jax
import jax.numpy as jnp
from jax import lax
from jax.experimental import pallas as pl
from jax.experimental.pallas import tpu as pltpu
from jax.experimental.pallas import tpu_sc as plsc

_PCODE = 2380
_NROW = 2 * (_PCODE + 1)
_D = 64
_B, _H, _W = 1024, 19, 19
_HW = _H * _W
_HWP = 368
_NV = _HWP // 16
_NC, _NS = 2, 16
_NW = _NC * _NS
_NG = 4
_DG = _D // _NG
_NSG = _NW // _NG
_SPT = _B // _NSG
_TILE = _DG * _HW


def _body(prep, table_r, out,
          table_v, in0_v, in1_v, outt0_v, outt1_v,
          sem_t, sem_in0, sem_in1, sem_out):
    wid = lax.axis_index("s") * _NC + lax.axis_index("c")
    g = wid % _NG
    sg = wid // _NG
    base = sg * _SPT
    lanes = lax.iota(jnp.int32, 16)
    tail_mask = lanes < (_HW - 16 * (_NV - 1))
    ins = (in0_v, in1_v)
    outs = (outt0_v, outt1_v)
    sems = (sem_in0, sem_in1)

    pltpu.async_copy(table_r.at[g], table_v, sem_t)
    pltpu.async_copy(prep.at[base], ins[0], sem_in0)
    pltpu.async_copy(prep.at[base + 1], ins[1], sem_in1)
    pltpu.make_async_copy(table_r.at[g], table_v, sem_t).wait()

    def compute(in_ref, out_ref):
        for i in range(_NV):
            sl = pl.ds(16 * i, 16)
            s0 = in_ref[0, sl]
            s1 = in_ref[1, sl]
            b0 = plsc.bitcast(in_ref[2, sl], jnp.float32)
            b1 = plsc.bitcast(in_ref[3, sl], jnp.float32)
            i0 = jnp.where(b0 > 0.0, _PCODE, s0)
            i1 = jnp.where(b1 > 0.0, _PCODE + _PCODE + 1, s1 + (_PCODE + 1))
            mask = tail_mask if i == _NV - 1 else None
            v0 = plsc.load_gather(table_v, [i0])
            v1 = plsc.load_gather(table_v, [i1])
            for d in range(1, _DG):
                n0 = plsc.load_gather(table_v, [i0 + d * _NROW])
                n1 = plsc.load_gather(table_v, [i1 + d * _NROW])
                plsc.store_scatter(out_ref,
                                   [lanes + ((d - 1) * _HW + 16 * i)],
                                   v0 + v1, mask=mask)
                v0, v1 = n0, n1
            plsc.store_scatter(out_ref,
                               [lanes + ((_DG - 1) * _HW + 16 * i)],
                               v0 + v1, mask=mask)

    @pl.loop(0, _SPT, step=2)
    def _pair(s0):
        for par in range(2):
            s = s0 + par
            b = base + s
            pltpu.make_async_copy(prep.at[b], ins[par], sems[par]).wait()
            compute(ins[par], outs[par])
            @pl.when(s > 0)
            def _():
                pltpu.make_async_copy(outs[par], out.at[b, g], sem_out).wait()
            pltpu.async_copy(outs[par], out.at[b, g], sem_out)
            pltpu.async_copy(prep.at[jnp.minimum(b + 2, _B - 1)],
                             ins[par], sems[par])

    pltpu.make_async_copy(outt0_v, out.at[base, g], sem_out).wait()
    pltpu.make_async_copy(prep.at[base], in0_v, sem_in0).wait()
    pltpu.make_async_copy(prep.at[base], in1_v, sem_in1).wait()


@jax.jit
def _pcode_embed(prep, table_r):
    mesh = plsc.VectorSubcoreMesh(core_axis_name="c", subcore_axis_name="s",
                                  num_cores=_NC, num_subcores=_NS)
    f = pl.kernel(
        _body,
        out_type=jax.ShapeDtypeStruct((_B, _NG, _TILE), jnp.float32),
        mesh=mesh,
        compiler_params=pltpu.CompilerParams(needs_layout_passes=False,
                                             use_tc_tiling_on_sc=False),
        scratch_types=[
            pltpu.VMEM((_NROW * _DG,), jnp.float32),
            pltpu.VMEM((4, _HWP), jnp.int32),
            pltpu.VMEM((4, _HWP), jnp.int32),
            pltpu.VMEM((_TILE,), jnp.float32),
            pltpu.VMEM((_TILE,), jnp.float32),
            pltpu.SemaphoreType.DMA,
            pltpu.SemaphoreType.DMA,
            pltpu.SemaphoreType.DMA,
            pltpu.SemaphoreType.DMA,
        ],
    )
    return f(prep, table_r)


def kernel(sparse_feature_input, board_input, sparse_feature_dim, pcode_table):
    del sparse_feature_dim
    pad = ((0, 0), (0, 0), (0, _HWP - _HW))
    sf = sparse_feature_input.reshape(_B, 12, _HW)[:, 10:12]
    bd = board_input.reshape(_B, 2, _HW).view(jnp.int32)
    prep = jnp.pad(jnp.concatenate([sf, bd], axis=1), pad)
    table_r = pcode_table.reshape(_NROW, _NG, _DG).transpose(1, 2, 0)
    table_r = table_r.reshape(_NG, _DG * _NROW)
    out = _pcode_embed(prep, table_r)
    return out.reshape(_B, _D, _H, _W)


# speedup vs baseline: 1.6402x; 1.1654x over previous
import jax
import jax.numpy as jnp
from jax import lax
from jax.experimental import pallas as pl
from jax.experimental.pallas import tpu as pltpu
from jax.experimental.pallas import tpu_sc as plsc

_PCODE = 2380
_NROW = 2 * (_PCODE + 1)
_D = 64
_B, _H, _W = 1024, 19, 19
_HW = _H * _W
_HWP = 368
_NV = _HWP // 16
_NC, _NS = 2, 16
_NW = _NC * _NS
_NG = 4
_DG = _D // _NG
_NSG = _NW // _NG
_SPT = _B // _NSG
_TILE = _DG * _HW


def _body(prep, table_r, out,
          table_v, in0_v, in1_v, outt0_v, outt1_v,
          sem_t, sem_in0, sem_in1, sem_out):
    wid = lax.axis_index("s") * _NC + lax.axis_index("c")
    g = wid % _NG
    sg = wid // _NG
    base = sg * _SPT
    lanes = lax.iota(jnp.int32, 16)
    tail_mask = lanes < (_HW - 16 * (_NV - 1))
    ins = (in0_v, in1_v)
    outs = (outt0_v, outt1_v)
    sems = (sem_in0, sem_in1)

    pltpu.async_copy(table_r.at[g], table_v, sem_t)
    pltpu.async_copy(prep.at[base], ins[0], sem_in0)
    pltpu.async_copy(prep.at[base + 1], ins[1], sem_in1)
    pltpu.make_async_copy(table_r.at[g], table_v, sem_t).wait()

    def compute(in_ref, out_ref):
        def idx_pair(i):
            sl = pl.ds(16 * i, 16)
            s0 = in_ref[0, sl]
            s1 = in_ref[1, sl]
            b0 = plsc.bitcast(in_ref[2, sl], jnp.float32)
            b1 = plsc.bitcast(in_ref[3, sl], jnp.float32)
            i0 = jnp.where(b0 > 0.0, _PCODE, s0)
            i1 = jnp.where(b1 > 0.0, _PCODE + _PCODE + 1, s1 + (_PCODE + 1))
            return i0, i1

        def gather_blocks(blocks, mask):
            idx = [idx_pair(i) for i in blocks]
            v = [plsc.load_gather(table_v, [ic]) for pair in idx
                 for ic in pair]
            for d in range(1, _DG):
                n = [plsc.load_gather(table_v, [ic + d * _NROW])
                     for pair in idx for ic in pair]
                for j, i in enumerate(blocks):
                    plsc.store_scatter(
                        out_ref, [lanes + ((d - 1) * _HW + 16 * i)],
                        v[2 * j] + v[2 * j + 1], mask=mask)
                v = n
            for j, i in enumerate(blocks):
                plsc.store_scatter(
                    out_ref, [lanes + ((_DG - 1) * _HW + 16 * i)],
                    v[2 * j] + v[2 * j + 1], mask=mask)

        for i in range(0, _NV - 1, 2):
            gather_blocks((i, i + 1), None)
        gather_blocks((_NV - 1,), tail_mask)

    @pl.loop(0, _SPT, step=2)
    def _pair(s0):
        for par in range(2):
            s = s0 + par
            b = base + s
            pltpu.make_async_copy(prep.at[b], ins[par], sems[par]).wait()
            compute(ins[par], outs[par])
            @pl.when(s > 0)
            def _():
                pltpu.make_async_copy(outs[par], out.at[b, g], sem_out).wait()
            pltpu.async_copy(outs[par], out.at[b, g], sem_out)
            pltpu.async_copy(prep.at[jnp.minimum(b + 2, _B - 1)],
                             ins[par], sems[par])

    pltpu.make_async_copy(outt0_v, out.at[base, g], sem_out).wait()
    pltpu.make_async_copy(prep.at[base], in0_v, sem_in0).wait()
    pltpu.make_async_copy(prep.at[base], in1_v, sem_in1).wait()


@jax.jit
def _pcode_embed(prep, table_r):
    mesh = plsc.VectorSubcoreMesh(core_axis_name="c", subcore_axis_name="s",
                                  num_cores=_NC, num_subcores=_NS)
    f = pl.kernel(
        _body,
        out_type=jax.ShapeDtypeStruct((_B, _NG, _TILE), jnp.float32),
        mesh=mesh,
        compiler_params=pltpu.CompilerParams(needs_layout_passes=False,
                                             use_tc_tiling_on_sc=False),
        scratch_types=[
            pltpu.VMEM((_NROW * _DG,), jnp.float32),
            pltpu.VMEM((4, _HWP), jnp.int32),
            pltpu.VMEM((4, _HWP), jnp.int32),
            pltpu.VMEM((_TILE,), jnp.float32),
            pltpu.VMEM((_TILE,), jnp.float32),
            pltpu.SemaphoreType.DMA,
            pltpu.SemaphoreType.DMA,
            pltpu.SemaphoreType.DMA,
            pltpu.SemaphoreType.DMA,
        ],
    )
    return f(prep, table_r)


def kernel(sparse_feature_input, board_input, sparse_feature_dim, pcode_table):
    del sparse_feature_dim
    pad = ((0, 0), (0, 0), (0, _HWP - _HW))
    sf = sparse_feature_input.reshape(_B, 12, _HW)[:, 10:12]
    bd = board_input.reshape(_B, 2, _HW).view(jnp.int32)
    prep = jnp.pad(jnp.concatenate([sf, bd], axis=1), pad)
    table_r = pcode_table.reshape(_NROW, _NG, _DG).transpose(1, 2, 0)
    table_r = table_r.reshape(_NG, _DG * _NROW)
    out = _pcode_embed(prep, table_r)
    return out.reshape(_B, _D, _H, _W)
